# baseline (speedup 1.0000x reference)
import functools

import jax
import jax.numpy as jnp
from jax import lax
from jax.experimental import pallas as pl
from jax.experimental.pallas import tpu as pltpu
from jax.experimental.pallas import tpu_sc as plsc

N_FACTORS = 32
BATCH = 16384

NC = 2
NS = 16
LANES = 16
NW = NC * NS
BPW = BATCH // NW
CHUNK = 128
NCHUNK = BPW // CHUNK
GROUPS = BPW // LANES


def _mf_kernel(u_hbm, m_hbm, U_hbm, M_hbm, out_hbm,
               idx_u, idx_m, rows_u, rows_m, out_v, sem):
    wid = lax.axis_index("s") * NC + lax.axis_index("c")
    base = wid * BPW

    pltpu.sync_copy(u_hbm.at[wid], idx_u)
    pltpu.sync_copy(m_hbm.at[wid], idx_m)

    copies = []
    for j in range(NCHUNK):
        copies.append(pltpu.async_copy(
            U_hbm.at[idx_u.at[j]], rows_u.at[pl.ds(j * CHUNK, CHUNK)], sem))
        copies.append(pltpu.async_copy(
            M_hbm.at[idx_m.at[j]], rows_m.at[pl.ds(j * CHUNK, CHUNK)], sem))
    for c in copies:
        c.wait()

    lanes = lax.iota(jnp.int32, LANES)

    def body(g, carry):
        row = jnp.full((LANES,), g * LANES, jnp.int32) + lanes
        acc = jnp.zeros((LANES,), jnp.float32)
        for d in range(N_FACTORS):
            col = jnp.full((LANES,), d, jnp.int32)
            uv = plsc.load_gather(rows_u, [row, col])
            mv = plsc.load_gather(rows_m, [row, col])
            acc = acc + uv * mv
        out_v[pl.ds(g * LANES, LANES)] = acc
        return carry

    lax.fori_loop(0, GROUPS, body, 0)

    pltpu.sync_copy(out_v, out_hbm.at[pl.ds(base, BPW)])


def kernel(users, movies, U, M):
    users3 = users.astype(jnp.int32).reshape(NW, NCHUNK, CHUNK)
    movies3 = movies.astype(jnp.int32).reshape(NW, NCHUNK, CHUNK)

    mesh = plsc.VectorSubcoreMesh(core_axis_name="c", subcore_axis_name="s")
    k = functools.partial(
        pl.kernel,
        mesh=mesh,
        compiler_params=pltpu.CompilerParams(
            needs_layout_passes=False, use_tc_tiling_on_sc=False),
        out_type=jax.ShapeDtypeStruct((BATCH,), jnp.float32),
        scratch_types=[
            pltpu.VMEM((NCHUNK, CHUNK), jnp.int32),
            pltpu.VMEM((NCHUNK, CHUNK), jnp.int32),
            pltpu.VMEM((BPW, N_FACTORS), jnp.float32),
            pltpu.VMEM((BPW, N_FACTORS), jnp.float32),
            pltpu.VMEM((BPW,), jnp.float32),
            pltpu.SemaphoreType.DMA,
        ],
    )(_mf_kernel)
    return k(users3, movies3, U, M)
